# baseline (speedup 1.0000x reference)
import functools

import jax
import jax.numpy as jnp
from jax import lax
from jax.experimental import pallas as pl
from jax.experimental.pallas import tpu as pltpu
from jax.experimental.pallas import tpu_sc as plsc

N_NODES_K = 10000
N_EDGES_K = 320000
QK = 50000

_NC = 2
_NS = 16
_NW = _NC * _NS


_NRP = 10240


def _edge_chunks(src, dst, n_edges, cw, nb):
    ch = -(-n_edges // (_NW * cw))
    ch = -(-ch // nb) * nb
    ep = _NW * cw * ch
    npad = ep - n_edges
    pad_src = jnp.arange(npad, dtype=jnp.int32) % N_NODES_K
    pad_dst = N_NODES_K + (jnp.arange(npad, dtype=jnp.int32) % (_NRP - N_NODES_K))
    srcp = jnp.concatenate([src, pad_src]).reshape(_NW, ch, cw)
    dstp = jnp.concatenate([dst, pad_dst]).reshape(_NW, ch, cw)
    return srcp, dstp, ch


def _degree_sc(dstp, ch):
    mesh = plsc.VectorSubcoreMesh(core_axis_name="c", subcore_axis_name="s")
    rows_per_tile = _NRP // _NS

    @functools.partial(
        pl.kernel, mesh=mesh,
        out_type=jax.ShapeDtypeStruct((_NC, _NRP, 16), jnp.float32),
        compiler_params=pltpu.CompilerParams(use_tc_tiling_on_sc=False),
        scratch_types=[
            pltpu.VMEM((ch, 128), jnp.int32),
            pltpu.VMEM((128, 16), jnp.float32),
            pltpu.VMEM_SHARED((_NRP, 16), jnp.float32),
        ])
    def k(dstp_hbm, out_hbm, di_v, ones_v, acc_sp):
        c = lax.axis_index("c")
        s = lax.axis_index("s")
        w = c * _NS + s
        pltpu.sync_copy(dstp_hbm.at[w], di_v)

        def zrow(r, carry):
            ones_v[r, :] = jnp.zeros((16,), jnp.float32)
            return carry

        lax.fori_loop(0, 128, zrow, 0)
        for b in range(rows_per_tile // 128):
            pltpu.sync_copy(ones_v, acc_sp.at[pl.ds(s * rows_per_tile + b * 128, 128)])

        def orow(r, carry):
            ones_v[r, :] = jnp.ones((16,), jnp.float32)
            return carry

        lax.fori_loop(0, 128, orow, 0)
        plsc.subcore_barrier()

        def chunk(j, carry):
            pltpu.sync_copy(ones_v, acc_sp.at[di_v.at[j]], add=True)
            return carry

        lax.fori_loop(0, ch, chunk, 0)
        plsc.subcore_barrier()
        pltpu.sync_copy(acc_sp.at[pl.ds(s * rows_per_tile, rows_per_tile)],
                        out_hbm.at[c, pl.ds(s * rows_per_tile, rows_per_tile)])

    return k(dstp)


def _msg_pass_sc(g, srcp, dstp, ch, d, cw, nb):
    mesh = plsc.VectorSubcoreMesh(core_axis_name="c", subcore_axis_name="s")
    rows_per_tile = _NRP // _NS
    h = nb // 2

    @functools.partial(
        pl.kernel, mesh=mesh,
        out_type=jax.ShapeDtypeStruct((_NC, _NRP, d), jnp.float32),
        compiler_params=pltpu.CompilerParams(use_tc_tiling_on_sc=False),
        scratch_types=[
            pltpu.VMEM((ch, cw), jnp.int32),
            pltpu.VMEM((ch, cw), jnp.int32),
            pltpu.VMEM((nb, cw, d), jnp.float32),
            pltpu.VMEM_SHARED((_NRP, d), jnp.float32),
            pltpu.SemaphoreType.DMA,
            pltpu.SemaphoreType.DMA,
        ])
    def k(g_hbm, srcp_hbm, dstp_hbm, out_hbm, si_v, di_v, rb_v, acc_sp, sa, sb):
        c = lax.axis_index("c")
        s = lax.axis_index("s")
        w = c * _NS + s
        pltpu.sync_copy(srcp_hbm.at[w], si_v)
        pltpu.sync_copy(dstp_hbm.at[w], di_v)

        def zrow(r, carry):
            for cc in range(d // 16):
                rb_v[0, r, pl.ds(cc * 16, 16)] = jnp.zeros((16,), jnp.float32)
            return carry

        lax.fori_loop(0, cw, zrow, 0)
        base = s * rows_per_tile
        for b in range(rows_per_tile // cw):
            pltpu.sync_copy(rb_v.at[0], acc_sp.at[pl.ds(base + b * cw, cw)])
        rem = rows_per_tile % cw
        if rem:
            pltpu.sync_copy(rb_v.at[0, pl.ds(0, rem)],
                            acc_sp.at[pl.ds(base + rows_per_tile - rem, rem)])
        plsc.subcore_barrier()

        def group(t, carry):
            j = nb * t
            cps_a = [pltpu.async_copy(g_hbm.at[si_v.at[j + i]], rb_v.at[i], sa)
                     for i in range(h)]
            for cp in cps_a:
                cp.wait()
            cps_b = [pltpu.async_copy(g_hbm.at[si_v.at[j + h + i]], rb_v.at[h + i], sb)
                     for i in range(nb - h)]
            for i in range(h):
                pltpu.sync_copy(rb_v.at[i], acc_sp.at[di_v.at[j + i]], add=True)
            for cp in cps_b:
                cp.wait()
            for i in range(nb - h):
                pltpu.sync_copy(rb_v.at[h + i], acc_sp.at[di_v.at[j + h + i]], add=True)
            return carry

        lax.fori_loop(0, ch // nb, group, 0)
        plsc.subcore_barrier()
        pltpu.sync_copy(acc_sp.at[pl.ds(s * rows_per_tile, rows_per_tile)],
                        out_hbm.at[c, pl.ds(s * rows_per_tile, rows_per_tile)])

    return k(g, srcp, dstp)


def _pair_sqdist_sc(emb, tsrc, tdst):
    q = tsrc.shape[0]
    ch = -(-q // (_NW * 128))
    qp = _NW * 128 * ch
    pad = jnp.arange(qp - q, dtype=jnp.int32) % N_NODES_K
    tsrcp = jnp.concatenate([tsrc, pad]).reshape(_NW, ch, 128)
    tdstp = jnp.concatenate([tdst, pad]).reshape(_NW, ch, 128)
    mesh = plsc.VectorSubcoreMesh(core_axis_name="c", subcore_axis_name="s")

    @functools.partial(
        pl.kernel, mesh=mesh,
        out_type=jax.ShapeDtypeStruct((_NW, ch, 128, 16), jnp.float32),
        compiler_params=pltpu.CompilerParams(use_tc_tiling_on_sc=False),
        scratch_types=[
            pltpu.VMEM((ch, 128), jnp.int32),
            pltpu.VMEM((ch, 128), jnp.int32),
            pltpu.VMEM((128, 16), jnp.float32),
            pltpu.VMEM((128, 16), jnp.float32),
            pltpu.SemaphoreType.DMA,
            pltpu.SemaphoreType.DMA,
        ])
    def k(emb_hbm, tsrc_hbm, tdst_hbm, out_hbm, ia_v, ib_v, ra_v, rb_v, sa, sb):
        c = lax.axis_index("c")
        s = lax.axis_index("s")
        w = c * _NS + s
        pltpu.sync_copy(tsrc_hbm.at[w], ia_v)
        pltpu.sync_copy(tdst_hbm.at[w], ib_v)

        def chunk(j, carry):
            ca = pltpu.async_copy(emb_hbm.at[ia_v.at[j]], ra_v, sa)
            cb = pltpu.async_copy(emb_hbm.at[ib_v.at[j]], rb_v, sb)
            ca.wait()
            cb.wait()

            def row(r, carry2):
                d = ra_v[r, :] - rb_v[r, :]
                ra_v[r, :] = d * d
                return carry2

            lax.fori_loop(0, 128, row, 0)
            pltpu.sync_copy(ra_v, out_hbm.at[w, j])
            return carry

        lax.fori_loop(0, ch, chunk, 0)

    out = k(emb, tsrcp, tdstp)
    return out.reshape(qp, 16)[:q]


def _decode_mlp_body(sq_ref, pi_ref, l1Wa_ref, l1Wb_ref, l1b_ref, lW_ref, lb_ref, out_ref):
    sq = sq_ref[...]
    pi = pi_ref[...]
    z = (jnp.dot(sq, l1Wa_ref[...], preferred_element_type=jnp.float32)
         + jnp.dot(pi, l1Wb_ref[...], preferred_element_type=jnp.float32)
         + l1b_ref[...])
    z = jnp.where(z >= 0, z, 0.2 * z)
    s = jnp.abs(jnp.dot(z, lW_ref[...], preferred_element_type=jnp.float32) + lb_ref[...])
    s = jnp.clip(s, 0.0, 40.0)
    out_ref[...] = 1.0 / (jnp.exp(s - 2.0) + 1.0)


def _decode_mlp(sqdist, PI_edges, l1W, l1b, lW, lb):
    q = sqdist.shape[0]
    blk = 5000
    grid = (q // blk,)
    out = pl.pallas_call(
        _decode_mlp_body,
        grid=grid,
        in_specs=[
            pl.BlockSpec((blk, 16), lambda i: (i, 0)),
            pl.BlockSpec((blk, 25), lambda i: (i, 0)),
            pl.BlockSpec((16, 25), lambda i: (0, 0)),
            pl.BlockSpec((25, 25), lambda i: (0, 0)),
            pl.BlockSpec((1, 25), lambda i: (0, 0)),
            pl.BlockSpec((25, 1), lambda i: (0, 0)),
            pl.BlockSpec((1, 1), lambda i: (0, 0)),
        ],
        out_specs=pl.BlockSpec((blk, 1), lambda i: (i, 0)),
        out_shape=jax.ShapeDtypeStruct((q, 1), jnp.float32),
    )(sqdist, PI_edges, l1W[:16], l1W[16:], l1b.reshape(1, 25), lW, lb.reshape(1, 1))
    return out.reshape(-1)


_RB = 1000


def _prep_body(x_ref, W1p_ref, dg0_ref, dg1_ref, g1_ref, dinv_ref):
    deg = dg0_ref[...][:, 0:1] + dg1_ref[...][:, 0:1] + 1.0
    dv = jax.lax.rsqrt(jnp.maximum(deg, 1e-12))
    g1_ref[...] = jnp.dot(x_ref[...], W1p_ref[...],
                          preferred_element_type=jnp.float32) * dv
    dinv_ref[...] = dv


def _prep_tc(x, W1p, degp):
    grid = (N_NODES_K // _RB,)
    return pl.pallas_call(
        _prep_body,
        grid=grid,
        in_specs=[
            pl.BlockSpec((_RB, 128), lambda i: (i, 0)),
            pl.BlockSpec((128, 112), lambda i: (0, 0)),
            pl.BlockSpec((_RB, 16), lambda i: (i, 0)),
            pl.BlockSpec((_RB, 16), lambda i: (i, 0)),
        ],
        out_specs=[
            pl.BlockSpec((_RB, 112), lambda i: (i, 0)),
            pl.BlockSpec((_RB, 1), lambda i: (i, 0)),
        ],
        out_shape=[
            jax.ShapeDtypeStruct((N_NODES_K, 112), jnp.float32),
            jax.ShapeDtypeStruct((N_NODES_K, 1), jnp.float32),
        ],
    )(x, W1p, degp[0], degp[1])


def _comb1_body(a0_ref, a1_ref, g1_ref, dinv_ref, b1p_ref, W2p_ref, g2_ref):
    dv = dinv_ref[...]
    h1 = dv * (a0_ref[...] + a1_ref[...] + g1_ref[...]) + b1p_ref[...]
    h1 = jnp.maximum(h1, 0.0)
    g2_ref[...] = jnp.dot(h1, W2p_ref[...],
                          preferred_element_type=jnp.float32) * dv


def _comb1_tc(accp, g1, dinv, b1p, W2p):
    grid = (N_NODES_K // _RB,)
    return pl.pallas_call(
        _comb1_body,
        grid=grid,
        in_specs=[
            pl.BlockSpec((_RB, 112), lambda i: (i, 0)),
            pl.BlockSpec((_RB, 112), lambda i: (i, 0)),
            pl.BlockSpec((_RB, 112), lambda i: (i, 0)),
            pl.BlockSpec((_RB, 1), lambda i: (i, 0)),
            pl.BlockSpec((1, 112), lambda i: (0, 0)),
            pl.BlockSpec((112, 16), lambda i: (0, 0)),
        ],
        out_specs=pl.BlockSpec((_RB, 16), lambda i: (i, 0)),
        out_shape=jax.ShapeDtypeStruct((N_NODES_K, 16), jnp.float32),
    )(accp[0], accp[1], g1, dinv, b1p, W2p)


def _comb2_body(a0_ref, a1_ref, g2_ref, dinv_ref, b2_ref, emb_ref):
    e = dinv_ref[...] * (a0_ref[...] + a1_ref[...] + g2_ref[...]) + b2_ref[...]
    e = jnp.maximum(e, 0.0)
    rn = jnp.sqrt(jnp.sum(e * e, axis=1, keepdims=True))
    scale = jnp.minimum(1.0, 1.0 / jnp.maximum(rn, 1e-7))
    emb_ref[...] = e * scale


def _comb2_tc(accp, g2, dinv, b2):
    grid = (N_NODES_K // _RB,)
    return pl.pallas_call(
        _comb2_body,
        grid=grid,
        in_specs=[
            pl.BlockSpec((_RB, 16), lambda i: (i, 0)),
            pl.BlockSpec((_RB, 16), lambda i: (i, 0)),
            pl.BlockSpec((_RB, 16), lambda i: (i, 0)),
            pl.BlockSpec((_RB, 1), lambda i: (i, 0)),
            pl.BlockSpec((1, 16), lambda i: (0, 0)),
        ],
        out_specs=pl.BlockSpec((_RB, 16), lambda i: (i, 0)),
        out_shape=jax.ShapeDtypeStruct((N_NODES_K, 16), jnp.float32),
    )(accp[0], accp[1], g2, dinv, b2)


def kernel(x, edge_index, total_edges, PI_edges, edges_y, W1, b1, W2, b2, l1W, l1b, lW, lb):
    src = jnp.asarray(edge_index[0], jnp.int32)
    dst = jnp.asarray(edge_index[1], jnp.int32)
    srcp1, dstp1, ch1 = _edge_chunks(src, dst, N_EDGES_K, 56, 6)
    srcp2, dstp2, ch2 = _edge_chunks(src, dst, N_EDGES_K, 128, 4)
    degp = _degree_sc(dstp2, ch2)
    W1p = jnp.pad(W1, ((0, 0), (0, 12)))
    b1p = jnp.pad(b1, (0, 12)).reshape(1, 112)
    W2p = jnp.pad(W2, ((0, 12), (0, 0)))
    g1, dinv = _prep_tc(x, W1p, degp)
    acc1p = _msg_pass_sc(g1, srcp1, dstp1, ch1, 112, 56, 6)
    g2 = _comb1_tc(acc1p, g1, dinv, b1p, W2p)
    acc2p = _msg_pass_sc(g2, srcp2, dstp2, ch2, 16, 128, 4)
    emb = _comb2_tc(acc2p, g2, dinv, b2.reshape(1, 16))
    tsrc = jnp.asarray(total_edges[:, 0], jnp.int32)
    tdst = jnp.asarray(total_edges[:, 1], jnp.int32)
    sqdist = _pair_sqdist_sc(emb, tsrc, tdst)
    prob = _decode_mlp(sqdist, PI_edges, l1W, l1b, lW, lb)
    return (prob, edges_y)


# speedup vs baseline: 1.0269x; 1.0269x over previous
import functools

import jax
import jax.numpy as jnp
from jax import lax
from jax.experimental import pallas as pl
from jax.experimental.pallas import tpu as pltpu
from jax.experimental.pallas import tpu_sc as plsc

N_NODES_K = 10000
N_EDGES_K = 320000
QK = 50000

_NC = 2
_NS = 16
_NW = _NC * _NS


_NRP = 10240


def _edge_chunks(src, dst, n_edges, cw, nb):
    ch = -(-n_edges // (_NW * cw))
    ch = -(-ch // nb) * nb
    ep = _NW * cw * ch
    npad = ep - n_edges
    pad_src = jnp.arange(npad, dtype=jnp.int32) % N_NODES_K
    pad_dst = N_NODES_K + (jnp.arange(npad, dtype=jnp.int32) % (_NRP - N_NODES_K))
    srcp = jnp.concatenate([src, pad_src]).reshape(_NW, ch, cw)
    dstp = jnp.concatenate([dst, pad_dst]).reshape(_NW, ch, cw)
    return srcp, dstp, ch


def _degree_sc(dstp, ch):
    mesh = plsc.VectorSubcoreMesh(core_axis_name="c", subcore_axis_name="s")
    rows_per_tile = _NRP // _NS

    @functools.partial(
        pl.kernel, mesh=mesh,
        out_type=jax.ShapeDtypeStruct((_NC, _NRP, 16), jnp.float32),
        compiler_params=pltpu.CompilerParams(use_tc_tiling_on_sc=False),
        scratch_types=[
            pltpu.VMEM((ch, 128), jnp.int32),
            pltpu.VMEM((128, 16), jnp.float32),
            pltpu.VMEM_SHARED((_NRP, 16), jnp.float32),
        ])
    def k(dstp_hbm, out_hbm, di_v, ones_v, acc_sp):
        c = lax.axis_index("c")
        s = lax.axis_index("s")
        w = c * _NS + s
        pltpu.sync_copy(dstp_hbm.at[w], di_v)

        def zrow(r, carry):
            ones_v[r, :] = jnp.zeros((16,), jnp.float32)
            return carry

        lax.fori_loop(0, 128, zrow, 0)
        for b in range(rows_per_tile // 128):
            pltpu.sync_copy(ones_v, acc_sp.at[pl.ds(s * rows_per_tile + b * 128, 128)])

        def orow(r, carry):
            ones_v[r, :] = jnp.ones((16,), jnp.float32)
            return carry

        lax.fori_loop(0, 128, orow, 0)
        plsc.subcore_barrier()

        def chunk(j, carry):
            pltpu.sync_copy(ones_v, acc_sp.at[di_v.at[j]], add=True)
            return carry

        lax.fori_loop(0, ch, chunk, 0)
        plsc.subcore_barrier()
        pltpu.sync_copy(acc_sp.at[pl.ds(s * rows_per_tile, rows_per_tile)],
                        out_hbm.at[c, pl.ds(s * rows_per_tile, rows_per_tile)])

    return k(dstp)


def _msg_pass_sc(g, srcp, dstp, ch, d, cw, nb, ovl):
    mesh = plsc.VectorSubcoreMesh(core_axis_name="c", subcore_axis_name="s")
    rows_per_tile = _NRP // _NS
    h = nb // 2

    @functools.partial(
        pl.kernel, mesh=mesh,
        out_type=jax.ShapeDtypeStruct((_NC, _NRP, d), jnp.float32),
        compiler_params=pltpu.CompilerParams(use_tc_tiling_on_sc=False),
        scratch_types=[
            pltpu.VMEM((ch, cw), jnp.int32),
            pltpu.VMEM((ch, cw), jnp.int32),
            pltpu.VMEM((nb, cw, d), jnp.float32),
            pltpu.VMEM_SHARED((_NRP, d), jnp.float32),
            pltpu.SemaphoreType.DMA,
            pltpu.SemaphoreType.DMA,
        ])
    def k(g_hbm, srcp_hbm, dstp_hbm, out_hbm, si_v, di_v, rb_v, acc_sp, sa, sb):
        c = lax.axis_index("c")
        s = lax.axis_index("s")
        w = c * _NS + s
        pltpu.sync_copy(srcp_hbm.at[w], si_v)
        pltpu.sync_copy(dstp_hbm.at[w], di_v)

        def zrow(r, carry):
            for cc in range(d // 16):
                rb_v[0, r, pl.ds(cc * 16, 16)] = jnp.zeros((16,), jnp.float32)
            return carry

        lax.fori_loop(0, cw, zrow, 0)
        base = s * rows_per_tile
        for b in range(rows_per_tile // cw):
            pltpu.sync_copy(rb_v.at[0], acc_sp.at[pl.ds(base + b * cw, cw)])
        rem = rows_per_tile % cw
        if rem:
            pltpu.sync_copy(rb_v.at[0, pl.ds(0, rem)],
                            acc_sp.at[pl.ds(base + rows_per_tile - rem, rem)])
        plsc.subcore_barrier()

        def group(t, carry):
            j = nb * t
            if ovl:
                cps_a = [pltpu.async_copy(g_hbm.at[si_v.at[j + i]], rb_v.at[i], sa)
                         for i in range(h)]
                for cp in cps_a:
                    cp.wait()
                cps_b = [pltpu.async_copy(g_hbm.at[si_v.at[j + h + i]], rb_v.at[h + i], sb)
                         for i in range(nb - h)]
                for i in range(h):
                    pltpu.sync_copy(rb_v.at[i], acc_sp.at[di_v.at[j + i]], add=True)
                for cp in cps_b:
                    cp.wait()
                for i in range(nb - h):
                    pltpu.sync_copy(rb_v.at[h + i], acc_sp.at[di_v.at[j + h + i]], add=True)
            else:
                cps = [pltpu.async_copy(g_hbm.at[si_v.at[j + i]], rb_v.at[i], sa)
                       for i in range(nb)]
                for cp in cps:
                    cp.wait()
                for i in range(nb):
                    pltpu.sync_copy(rb_v.at[i], acc_sp.at[di_v.at[j + i]], add=True)
            return carry

        lax.fori_loop(0, ch // nb, group, 0)
        plsc.subcore_barrier()
        pltpu.sync_copy(acc_sp.at[pl.ds(s * rows_per_tile, rows_per_tile)],
                        out_hbm.at[c, pl.ds(s * rows_per_tile, rows_per_tile)])

    return k(g, srcp, dstp)


def _emb_pairs_sc(acc2p, g2, dinv16, b2r, tsrc, tdst):
    q = tsrc.shape[0]
    ch = -(-q // (_NW * 128))
    qp = _NW * 128 * ch
    pad = jnp.arange(qp - q, dtype=jnp.int32) % N_NODES_K
    tsrcp = jnp.concatenate([tsrc, pad]).reshape(_NW, ch, 128)
    tdstp = jnp.concatenate([tdst, pad]).reshape(_NW, ch, 128)
    mesh = plsc.VectorSubcoreMesh(core_axis_name="c", subcore_axis_name="s")
    rpt = N_NODES_K // _NS

    @functools.partial(
        pl.kernel, mesh=mesh,
        out_type=jax.ShapeDtypeStruct((_NW, ch, 128, 16), jnp.float32),
        compiler_params=pltpu.CompilerParams(use_tc_tiling_on_sc=False),
        scratch_types=[
            pltpu.VMEM((rpt, 16), jnp.float32),
            pltpu.VMEM((rpt, 16), jnp.float32),
            pltpu.VMEM((rpt, 16), jnp.float32),
            pltpu.VMEM((rpt, 16), jnp.float32),
            pltpu.VMEM((1, 16), jnp.float32),
            pltpu.VMEM_SHARED((N_NODES_K, 16), jnp.float32),
            pltpu.VMEM((ch, 128), jnp.int32),
            pltpu.VMEM((ch, 128), jnp.int32),
            pltpu.VMEM((128, 16), jnp.float32),
            pltpu.VMEM((128, 16), jnp.float32),
            pltpu.SemaphoreType.DMA,
            pltpu.SemaphoreType.DMA,
        ])
    def k(a0_hbm, a1_hbm, g2_hbm, dv_hbm, b2_hbm, tsrc_hbm, tdst_hbm, out_hbm,
          a0_v, a1_v, g2_v, dv_v, b2_v, emb_sp, ia_v, ib_v, ra_v, rb_v, sa, sb):
        c = lax.axis_index("c")
        s = lax.axis_index("s")
        w = c * _NS + s
        base = s * rpt
        pltpu.sync_copy(a0_hbm.at[pl.ds(base, rpt)], a0_v)
        pltpu.sync_copy(a1_hbm.at[pl.ds(base, rpt)], a1_v)
        pltpu.sync_copy(g2_hbm.at[pl.ds(base, rpt)], g2_v)
        pltpu.sync_copy(dv_hbm.at[pl.ds(base, rpt)], dv_v)
        pltpu.sync_copy(b2_hbm, b2_v)
        pltpu.sync_copy(tsrc_hbm.at[w], ia_v)
        pltpu.sync_copy(tdst_hbm.at[w], ib_v)

        lanes = lax.iota(jnp.int32, 16)

        def group(gi, carry):
            r0 = gi * 16
            packed = jnp.zeros((16,), jnp.float32)
            for kk in range(16):
                r = r0 + kk
                e = dv_v[r, :] * (a0_v[r, :] + a1_v[r, :] + g2_v[r, :]) + b2_v[0, :]
                e = jnp.maximum(e, 0.0)
                a0_v[r, :] = e
                rn2 = e * e
                for st in (8, 4, 2, 1):
                    rn2 = rn2 + rn2.at[lanes ^ st].get(mode="promise_in_bounds")
                packed = jnp.where(lanes == kk, rn2, packed)
            xx = jnp.maximum(packed, 1.0)
            y = 1.0 / xx
            xh = 0.5 * xx
            for _ in range(18):
                y = y * (1.5 - xh * y * y)
            scale = jnp.where(packed > 1.0, y, 1.0)
            for kk in range(16):
                r = r0 + kk
                sk = scale.at[jnp.full((16,), kk, jnp.int32)].get(
                    mode="promise_in_bounds")
                a0_v[r, :] = a0_v[r, :] * sk
            return carry

        lax.fori_loop(0, rpt // 16, group, 0)
        e = dv_v[rpt - 1, :] * (a0_v[rpt - 1, :] + a1_v[rpt - 1, :]
                                + g2_v[rpt - 1, :]) + b2_v[0, :]
        e = jnp.maximum(e, 0.0)
        rn2 = e * e
        for st in (8, 4, 2, 1):
            rn2 = rn2 + rn2.at[lanes ^ st].get(mode="promise_in_bounds")
        xx = jnp.maximum(rn2, 1.0)
        y = 1.0 / xx
        xh = 0.5 * xx
        for _ in range(18):
            y = y * (1.5 - xh * y * y)
        a0_v[rpt - 1, :] = e * jnp.where(rn2 > 1.0, y, 1.0)
        pltpu.sync_copy(a0_v, emb_sp.at[pl.ds(base, rpt)])
        plsc.subcore_barrier()

        def chunk(j, carry):
            ca = pltpu.async_copy(emb_sp.at[ia_v.at[j]], ra_v, sa)
            cb = pltpu.async_copy(emb_sp.at[ib_v.at[j]], rb_v, sb)
            ca.wait()
            cb.wait()

            def prow(r, carry2):
                d = ra_v[r, :] - rb_v[r, :]
                ra_v[r, :] = d * d
                return carry2

            lax.fori_loop(0, 128, prow, 0)
            pltpu.sync_copy(ra_v, out_hbm.at[w, j])
            return carry

        lax.fori_loop(0, ch, chunk, 0)

    out = k(acc2p[0], acc2p[1], g2, dinv16, b2r, tsrcp, tdstp)
    return out.reshape(qp, 16)[:q]


def _decode_mlp_body(sq_ref, pi_ref, l1Wa_ref, l1Wb_ref, l1b_ref, lW_ref, lb_ref, out_ref):
    sq = sq_ref[...]
    pi = pi_ref[...]
    z = (jnp.dot(sq, l1Wa_ref[...], preferred_element_type=jnp.float32)
         + jnp.dot(pi, l1Wb_ref[...], preferred_element_type=jnp.float32)
         + l1b_ref[...])
    z = jnp.where(z >= 0, z, 0.2 * z)
    s = jnp.abs(jnp.dot(z, lW_ref[...], preferred_element_type=jnp.float32) + lb_ref[...])
    s = jnp.clip(s, 0.0, 40.0)
    out_ref[...] = 1.0 / (jnp.exp(s - 2.0) + 1.0)


def _decode_mlp(sqdist, PI_edges, l1W, l1b, lW, lb):
    q = sqdist.shape[0]
    blk = 5000
    grid = (q // blk,)
    out = pl.pallas_call(
        _decode_mlp_body,
        grid=grid,
        in_specs=[
            pl.BlockSpec((blk, 16), lambda i: (i, 0)),
            pl.BlockSpec((blk, 25), lambda i: (i, 0)),
            pl.BlockSpec((16, 25), lambda i: (0, 0)),
            pl.BlockSpec((25, 25), lambda i: (0, 0)),
            pl.BlockSpec((1, 25), lambda i: (0, 0)),
            pl.BlockSpec((25, 1), lambda i: (0, 0)),
            pl.BlockSpec((1, 1), lambda i: (0, 0)),
        ],
        out_specs=pl.BlockSpec((blk, 1), lambda i: (i, 0)),
        out_shape=jax.ShapeDtypeStruct((q, 1), jnp.float32),
    )(sqdist, PI_edges, l1W[:16], l1W[16:], l1b.reshape(1, 25), lW, lb.reshape(1, 1))
    return out.reshape(-1)


_RB = 1000


def _prep_body(x_ref, W1p_ref, dg0_ref, dg1_ref, g1_ref, dinv_ref, dinv16_ref):
    deg = dg0_ref[...][:, 0:1] + dg1_ref[...][:, 0:1] + 1.0
    dv = jax.lax.rsqrt(jnp.maximum(deg, 1e-12))
    g1_ref[...] = jnp.dot(x_ref[...], W1p_ref[...],
                          preferred_element_type=jnp.float32) * dv
    dinv_ref[...] = dv
    dinv16_ref[...] = jnp.broadcast_to(dv, (dv.shape[0], 16))


def _prep_tc(x, W1p, degp):
    grid = (N_NODES_K // _RB,)
    return pl.pallas_call(
        _prep_body,
        grid=grid,
        in_specs=[
            pl.BlockSpec((_RB, 128), lambda i: (i, 0)),
            pl.BlockSpec((128, 112), lambda i: (0, 0)),
            pl.BlockSpec((_RB, 16), lambda i: (i, 0)),
            pl.BlockSpec((_RB, 16), lambda i: (i, 0)),
        ],
        out_specs=[
            pl.BlockSpec((_RB, 112), lambda i: (i, 0)),
            pl.BlockSpec((_RB, 1), lambda i: (i, 0)),
            pl.BlockSpec((_RB, 16), lambda i: (i, 0)),
        ],
        out_shape=[
            jax.ShapeDtypeStruct((N_NODES_K, 112), jnp.float32),
            jax.ShapeDtypeStruct((N_NODES_K, 1), jnp.float32),
            jax.ShapeDtypeStruct((N_NODES_K, 16), jnp.float32),
        ],
    )(x, W1p, degp[0], degp[1])


def _comb1_body(a0_ref, a1_ref, g1_ref, dinv_ref, b1p_ref, W2p_ref, g2_ref):
    dv = dinv_ref[...]
    h1 = dv * (a0_ref[...] + a1_ref[...] + g1_ref[...]) + b1p_ref[...]
    h1 = jnp.maximum(h1, 0.0)
    g2_ref[...] = jnp.dot(h1, W2p_ref[...],
                          preferred_element_type=jnp.float32) * dv


def _comb1_tc(accp, g1, dinv, b1p, W2p):
    grid = (N_NODES_K // _RB,)
    return pl.pallas_call(
        _comb1_body,
        grid=grid,
        in_specs=[
            pl.BlockSpec((_RB, 112), lambda i: (i, 0)),
            pl.BlockSpec((_RB, 112), lambda i: (i, 0)),
            pl.BlockSpec((_RB, 112), lambda i: (i, 0)),
            pl.BlockSpec((_RB, 1), lambda i: (i, 0)),
            pl.BlockSpec((1, 112), lambda i: (0, 0)),
            pl.BlockSpec((112, 16), lambda i: (0, 0)),
        ],
        out_specs=pl.BlockSpec((_RB, 16), lambda i: (i, 0)),
        out_shape=jax.ShapeDtypeStruct((N_NODES_K, 16), jnp.float32),
    )(accp[0], accp[1], g1, dinv, b1p, W2p)


def kernel(x, edge_index, total_edges, PI_edges, edges_y, W1, b1, W2, b2, l1W, l1b, lW, lb):
    src = jnp.asarray(edge_index[0], jnp.int32)
    dst = jnp.asarray(edge_index[1], jnp.int32)
    srcp1, dstp1, ch1 = _edge_chunks(src, dst, N_EDGES_K, 56, 6)
    srcp2, dstp2, ch2 = _edge_chunks(src, dst, N_EDGES_K, 128, 4)
    degp = _degree_sc(dstp2, ch2)
    W1p = jnp.pad(W1, ((0, 0), (0, 12)))
    b1p = jnp.pad(b1, (0, 12)).reshape(1, 112)
    W2p = jnp.pad(W2, ((0, 12), (0, 0)))
    g1, dinv, dinv16 = _prep_tc(x, W1p, degp)
    acc1p = _msg_pass_sc(g1, srcp1, dstp1, ch1, 112, 56, 6, True)
    g2 = _comb1_tc(acc1p, g1, dinv, b1p, W2p)
    acc2p = _msg_pass_sc(g2, srcp2, dstp2, ch2, 16, 128, 4, False)
    tsrc = jnp.asarray(total_edges[:, 0], jnp.int32)
    tdst = jnp.asarray(total_edges[:, 1], jnp.int32)
    sqdist = _emb_pairs_sc(acc2p, g2, dinv16, b2.reshape(1, 16), tsrc, tdst)
    prob = _decode_mlp(sqdist, PI_edges, l1W, l1b, lW, lb)
    return (prob, edges_y)


# speedup vs baseline: 1.0428x; 1.0155x over previous
import functools

import jax
import jax.numpy as jnp
from jax import lax
from jax.experimental import pallas as pl
from jax.experimental.pallas import tpu as pltpu
from jax.experimental.pallas import tpu_sc as plsc

N_NODES_K = 10000
N_EDGES_K = 320000
QK = 50000

_NC = 2
_NS = 16
_NW = _NC * _NS


_NRP = 10240


def _edge_chunks(src, dst, n_edges, cw, nb):
    ch = -(-n_edges // (_NW * cw))
    ch = -(-ch // nb) * nb
    ep = _NW * cw * ch
    npad = ep - n_edges
    pad_src = jnp.arange(npad, dtype=jnp.int32) % N_NODES_K
    pad_dst = N_NODES_K + (jnp.arange(npad, dtype=jnp.int32) % (_NRP - N_NODES_K))
    srcp = jnp.concatenate([src, pad_src]).reshape(_NW, ch, cw)
    dstp = jnp.concatenate([dst, pad_dst]).reshape(_NW, ch, cw)
    return srcp, dstp, ch


def _degree_sc(dstp, ch):
    mesh = plsc.VectorSubcoreMesh(core_axis_name="c", subcore_axis_name="s")
    rows_per_tile = _NRP // _NS

    @functools.partial(
        pl.kernel, mesh=mesh,
        out_type=jax.ShapeDtypeStruct((_NC, _NRP, 16), jnp.float32),
        compiler_params=pltpu.CompilerParams(use_tc_tiling_on_sc=False),
        scratch_types=[
            pltpu.VMEM((ch, 128), jnp.int32),
            pltpu.VMEM((128, 16), jnp.float32),
            pltpu.VMEM_SHARED((_NRP, 16), jnp.float32),
        ])
    def k(dstp_hbm, out_hbm, di_v, ones_v, acc_sp):
        c = lax.axis_index("c")
        s = lax.axis_index("s")
        w = c * _NS + s
        pltpu.sync_copy(dstp_hbm.at[w], di_v)

        def zrow(r, carry):
            ones_v[r, :] = jnp.zeros((16,), jnp.float32)
            return carry

        lax.fori_loop(0, 128, zrow, 0)
        for b in range(rows_per_tile // 128):
            pltpu.sync_copy(ones_v, acc_sp.at[pl.ds(s * rows_per_tile + b * 128, 128)])

        def orow(r, carry):
            ones_v[r, :] = jnp.ones((16,), jnp.float32)
            return carry

        lax.fori_loop(0, 128, orow, 0)
        plsc.subcore_barrier()

        def chunk(j, carry):
            pltpu.sync_copy(ones_v, acc_sp.at[di_v.at[j]], add=True)
            return carry

        lax.fori_loop(0, ch, chunk, 0)
        plsc.subcore_barrier()
        pltpu.sync_copy(acc_sp.at[pl.ds(s * rows_per_tile, rows_per_tile)],
                        out_hbm.at[c, pl.ds(s * rows_per_tile, rows_per_tile)])

    return k(dstp)


def _msg_pass_sc(g, srcp, dstp, ch, d, cw, nb, ovl):
    mesh = plsc.VectorSubcoreMesh(core_axis_name="c", subcore_axis_name="s")
    rows_per_tile = _NRP // _NS
    h = nb // 2

    @functools.partial(
        pl.kernel, mesh=mesh,
        out_type=jax.ShapeDtypeStruct((_NC, _NRP, d), jnp.float32),
        compiler_params=pltpu.CompilerParams(use_tc_tiling_on_sc=False),
        scratch_types=[
            pltpu.VMEM((ch, cw), jnp.int32),
            pltpu.VMEM((ch, cw), jnp.int32),
            pltpu.VMEM((nb, cw, d), jnp.float32),
            pltpu.VMEM_SHARED((_NRP, d), jnp.float32),
            pltpu.SemaphoreType.DMA,
            pltpu.SemaphoreType.DMA,
        ])
    def k(g_hbm, srcp_hbm, dstp_hbm, out_hbm, si_v, di_v, rb_v, acc_sp, sa, sb):
        c = lax.axis_index("c")
        s = lax.axis_index("s")
        w = c * _NS + s
        pltpu.sync_copy(srcp_hbm.at[w], si_v)
        pltpu.sync_copy(dstp_hbm.at[w], di_v)

        def zrow(r, carry):
            for cc in range(d // 16):
                rb_v[0, r, pl.ds(cc * 16, 16)] = jnp.zeros((16,), jnp.float32)
            return carry

        lax.fori_loop(0, cw, zrow, 0)
        base = s * rows_per_tile
        for b in range(rows_per_tile // cw):
            pltpu.sync_copy(rb_v.at[0], acc_sp.at[pl.ds(base + b * cw, cw)])
        rem = rows_per_tile % cw
        if rem:
            pltpu.sync_copy(rb_v.at[0, pl.ds(0, rem)],
                            acc_sp.at[pl.ds(base + rows_per_tile - rem, rem)])
        plsc.subcore_barrier()

        def group(t, carry):
            j = nb * t
            if ovl:
                cps_a = [pltpu.async_copy(g_hbm.at[si_v.at[j + i]], rb_v.at[i], sa)
                         for i in range(h)]
                for cp in cps_a:
                    cp.wait()
                cps_b = [pltpu.async_copy(g_hbm.at[si_v.at[j + h + i]], rb_v.at[h + i], sb)
                         for i in range(nb - h)]
                for i in range(h):
                    pltpu.sync_copy(rb_v.at[i], acc_sp.at[di_v.at[j + i]], add=True)
                for cp in cps_b:
                    cp.wait()
                for i in range(nb - h):
                    pltpu.sync_copy(rb_v.at[h + i], acc_sp.at[di_v.at[j + h + i]], add=True)
            else:
                cps = [pltpu.async_copy(g_hbm.at[si_v.at[j + i]], rb_v.at[i], sa)
                       for i in range(nb)]
                for cp in cps:
                    cp.wait()
                for i in range(nb):
                    pltpu.sync_copy(rb_v.at[i], acc_sp.at[di_v.at[j + i]], add=True)
            return carry

        lax.fori_loop(0, ch // nb, group, 0)
        plsc.subcore_barrier()
        pltpu.sync_copy(acc_sp.at[pl.ds(s * rows_per_tile, rows_per_tile)],
                        out_hbm.at[c, pl.ds(s * rows_per_tile, rows_per_tile)])

    return k(g, srcp, dstp)


def _emb_pairs_sc(acc2p, g2, dinv16, b2r, tsrc, tdst):
    q = tsrc.shape[0]
    ch = -(-q // (_NW * 128))
    qp = _NW * 128 * ch
    pad = jnp.arange(qp - q, dtype=jnp.int32) % N_NODES_K
    tsrcp = jnp.concatenate([tsrc, pad]).reshape(_NW, ch, 128)
    tdstp = jnp.concatenate([tdst, pad]).reshape(_NW, ch, 128)
    mesh = plsc.VectorSubcoreMesh(core_axis_name="c", subcore_axis_name="s")
    rpt = N_NODES_K // _NS

    @functools.partial(
        pl.kernel, mesh=mesh,
        out_type=jax.ShapeDtypeStruct((_NW, ch, 128, 16), jnp.float32),
        compiler_params=pltpu.CompilerParams(use_tc_tiling_on_sc=False),
        scratch_types=[
            pltpu.VMEM((rpt, 16), jnp.float32),
            pltpu.VMEM((rpt, 16), jnp.float32),
            pltpu.VMEM((rpt, 16), jnp.float32),
            pltpu.VMEM((rpt, 16), jnp.float32),
            pltpu.VMEM((1, 16), jnp.float32),
            pltpu.VMEM_SHARED((N_NODES_K, 16), jnp.float32),
            pltpu.VMEM((ch, 128), jnp.int32),
            pltpu.VMEM((ch, 128), jnp.int32),
            pltpu.VMEM((128, 16), jnp.float32),
            pltpu.VMEM((128, 16), jnp.float32),
            pltpu.SemaphoreType.DMA,
            pltpu.SemaphoreType.DMA,
        ])
    def k(a0_hbm, a1_hbm, g2_hbm, dv_hbm, b2_hbm, tsrc_hbm, tdst_hbm, out_hbm,
          a0_v, a1_v, g2_v, dv_v, b2_v, emb_sp, ia_v, ib_v, ra_v, rb_v, sa, sb):
        c = lax.axis_index("c")
        s = lax.axis_index("s")
        w = c * _NS + s
        base = s * rpt
        pltpu.sync_copy(a0_hbm.at[pl.ds(base, rpt)], a0_v)
        pltpu.sync_copy(a1_hbm.at[pl.ds(base, rpt)], a1_v)
        pltpu.sync_copy(g2_hbm.at[pl.ds(base, rpt)], g2_v)
        pltpu.sync_copy(dv_hbm.at[pl.ds(base, rpt)], dv_v)
        pltpu.sync_copy(b2_hbm, b2_v)
        pltpu.sync_copy(tsrc_hbm.at[w], ia_v)
        pltpu.sync_copy(tdst_hbm.at[w], ib_v)

        lanes = lax.iota(jnp.int32, 16)

        def group(gi, carry):
            r0 = gi * 16
            packed = jnp.zeros((16,), jnp.float32)
            for kk in range(16):
                r = r0 + kk
                e = dv_v[r, :] * (a0_v[r, :] + a1_v[r, :] + g2_v[r, :]) + b2_v[0, :]
                e = jnp.maximum(e, 0.0)
                a0_v[r, :] = e
                rn2 = e * e
                for st in (8, 4, 2, 1):
                    rn2 = rn2 + rn2.at[lanes ^ st].get(mode="promise_in_bounds")
                packed = jnp.where(lanes == kk, rn2, packed)
            xx = jnp.maximum(packed, 1.0)
            y = 1.0 / xx
            xh = 0.5 * xx
            for _ in range(18):
                y = y * (1.5 - xh * y * y)
            scale = jnp.where(packed > 1.0, y, 1.0)
            for kk in range(16):
                r = r0 + kk
                sk = scale.at[jnp.full((16,), kk, jnp.int32)].get(
                    mode="promise_in_bounds")
                a0_v[r, :] = a0_v[r, :] * sk
            return carry

        lax.fori_loop(0, rpt // 16, group, 0)
        e = dv_v[rpt - 1, :] * (a0_v[rpt - 1, :] + a1_v[rpt - 1, :]
                                + g2_v[rpt - 1, :]) + b2_v[0, :]
        e = jnp.maximum(e, 0.0)
        rn2 = e * e
        for st in (8, 4, 2, 1):
            rn2 = rn2 + rn2.at[lanes ^ st].get(mode="promise_in_bounds")
        xx = jnp.maximum(rn2, 1.0)
        y = 1.0 / xx
        xh = 0.5 * xx
        for _ in range(18):
            y = y * (1.5 - xh * y * y)
        a0_v[rpt - 1, :] = e * jnp.where(rn2 > 1.0, y, 1.0)
        pltpu.sync_copy(a0_v, emb_sp.at[pl.ds(base, rpt)])
        plsc.subcore_barrier()

        def chunk(j, carry):
            ca = pltpu.async_copy(emb_sp.at[ia_v.at[j]], ra_v, sa)
            cb = pltpu.async_copy(emb_sp.at[ib_v.at[j]], rb_v, sb)
            ca.wait()
            cb.wait()

            def prow(r, carry2):
                d = ra_v[r, :] - rb_v[r, :]
                ra_v[r, :] = d * d
                return carry2

            lax.fori_loop(0, 128, prow, 0)
            pltpu.sync_copy(ra_v, out_hbm.at[w, j])
            return carry

        lax.fori_loop(0, ch, chunk, 0)

    out = k(acc2p[0], acc2p[1], g2, dinv16, b2r, tsrcp, tdstp)
    return out.reshape(qp, 16)[:q]


def _decode_mlp_body(sq_ref, pi_ref, l1Wa_ref, l1Wb_ref, l1b_ref, lW_ref, lb_ref, out_ref):
    sq = sq_ref[...]
    pi = pi_ref[...]
    z = (jnp.dot(sq, l1Wa_ref[...], preferred_element_type=jnp.float32)
         + jnp.dot(pi, l1Wb_ref[...], preferred_element_type=jnp.float32)
         + l1b_ref[...])
    z = jnp.where(z >= 0, z, 0.2 * z)
    s = jnp.abs(jnp.dot(z, lW_ref[...], preferred_element_type=jnp.float32) + lb_ref[...])
    s = jnp.clip(s, 0.0, 40.0)
    out_ref[...] = 1.0 / (jnp.exp(s - 2.0) + 1.0)


def _decode_mlp(sqdist, PI_edges, l1W, l1b, lW, lb):
    q = sqdist.shape[0]
    blk = 5000
    grid = (q // blk,)
    out = pl.pallas_call(
        _decode_mlp_body,
        grid=grid,
        in_specs=[
            pl.BlockSpec((blk, 16), lambda i: (i, 0)),
            pl.BlockSpec((blk, 25), lambda i: (i, 0)),
            pl.BlockSpec((16, 25), lambda i: (0, 0)),
            pl.BlockSpec((25, 25), lambda i: (0, 0)),
            pl.BlockSpec((1, 25), lambda i: (0, 0)),
            pl.BlockSpec((25, 1), lambda i: (0, 0)),
            pl.BlockSpec((1, 1), lambda i: (0, 0)),
        ],
        out_specs=pl.BlockSpec((blk, 1), lambda i: (i, 0)),
        out_shape=jax.ShapeDtypeStruct((q, 1), jnp.float32),
    )(sqdist, PI_edges, l1W[:16], l1W[16:], l1b.reshape(1, 25), lW, lb.reshape(1, 1))
    return out.reshape(-1)


_RB = 1000


def _prep_body(x_ref, W1p_ref, dg0_ref, dg1_ref, g1_ref, dinv_ref, dinv16_ref):
    deg = dg0_ref[...][:, 0:1] + dg1_ref[...][:, 0:1] + 1.0
    dv = jax.lax.rsqrt(jnp.maximum(deg, 1e-12))
    g1_ref[...] = jnp.dot(x_ref[...], W1p_ref[...],
                          preferred_element_type=jnp.float32) * dv
    dinv_ref[...] = dv
    dinv16_ref[...] = jnp.broadcast_to(dv, (dv.shape[0], 16))


def _prep_tc(x, W1p, degp):
    grid = (N_NODES_K // _RB,)
    return pl.pallas_call(
        _prep_body,
        grid=grid,
        in_specs=[
            pl.BlockSpec((_RB, 128), lambda i: (i, 0)),
            pl.BlockSpec((128, 112), lambda i: (0, 0)),
            pl.BlockSpec((_RB, 16), lambda i: (i, 0)),
            pl.BlockSpec((_RB, 16), lambda i: (i, 0)),
        ],
        out_specs=[
            pl.BlockSpec((_RB, 112), lambda i: (i, 0)),
            pl.BlockSpec((_RB, 1), lambda i: (i, 0)),
            pl.BlockSpec((_RB, 16), lambda i: (i, 0)),
        ],
        out_shape=[
            jax.ShapeDtypeStruct((N_NODES_K, 112), jnp.float32),
            jax.ShapeDtypeStruct((N_NODES_K, 1), jnp.float32),
            jax.ShapeDtypeStruct((N_NODES_K, 16), jnp.float32),
        ],
    )(x, W1p, degp[0], degp[1])


def _comb1_body(a0_ref, a1_ref, g1_ref, dinv_ref, b1p_ref, W2p_ref, g2_ref):
    dv = dinv_ref[...]
    h1 = dv * (a0_ref[...] + a1_ref[...] + g1_ref[...]) + b1p_ref[...]
    h1 = jnp.maximum(h1, 0.0)
    g2_ref[...] = jnp.dot(h1, W2p_ref[...],
                          preferred_element_type=jnp.float32) * dv


def _comb1_tc(accp, g1, dinv, b1p, W2p):
    grid = (N_NODES_K // _RB,)
    return pl.pallas_call(
        _comb1_body,
        grid=grid,
        in_specs=[
            pl.BlockSpec((_RB, 112), lambda i: (i, 0)),
            pl.BlockSpec((_RB, 112), lambda i: (i, 0)),
            pl.BlockSpec((_RB, 112), lambda i: (i, 0)),
            pl.BlockSpec((_RB, 1), lambda i: (i, 0)),
            pl.BlockSpec((1, 112), lambda i: (0, 0)),
            pl.BlockSpec((112, 16), lambda i: (0, 0)),
        ],
        out_specs=pl.BlockSpec((_RB, 16), lambda i: (i, 0)),
        out_shape=jax.ShapeDtypeStruct((N_NODES_K, 16), jnp.float32),
    )(accp[0], accp[1], g1, dinv, b1p, W2p)


def kernel(x, edge_index, total_edges, PI_edges, edges_y, W1, b1, W2, b2, l1W, l1b, lW, lb):
    src = jnp.asarray(edge_index[0], jnp.int32)
    dst = jnp.asarray(edge_index[1], jnp.int32)
    srcp1, dstp1, ch1 = _edge_chunks(src, dst, N_EDGES_K, 56, 6)
    srcp2, dstp2, ch2 = _edge_chunks(src, dst, N_EDGES_K, 128, 8)
    degp = _degree_sc(dstp2, ch2)
    W1p = jnp.pad(W1, ((0, 0), (0, 12)))
    b1p = jnp.pad(b1, (0, 12)).reshape(1, 112)
    W2p = jnp.pad(W2, ((0, 12), (0, 0)))
    g1, dinv, dinv16 = _prep_tc(x, W1p, degp)
    acc1p = _msg_pass_sc(g1, srcp1, dstp1, ch1, 112, 56, 6, True)
    g2 = _comb1_tc(acc1p, g1, dinv, b1p, W2p)
    acc2p = _msg_pass_sc(g2, srcp2, dstp2, ch2, 16, 128, 8, False)
    tsrc = jnp.asarray(total_edges[:, 0], jnp.int32)
    tdst = jnp.asarray(total_edges[:, 1], jnp.int32)
    sqdist = _emb_pairs_sc(acc2p, g2, dinv16, b2.reshape(1, 16), tsrc, tdst)
    prob = _decode_mlp(sqdist, PI_edges, l1W, l1b, lW, lb)
    return (prob, edges_y)


# speedup vs baseline: 1.0446x; 1.0017x over previous
import functools

import jax
import jax.numpy as jnp
from jax import lax
from jax.experimental import pallas as pl
from jax.experimental.pallas import tpu as pltpu
from jax.experimental.pallas import tpu_sc as plsc

N_NODES_K = 10000
N_EDGES_K = 320000
QK = 50000

_NC = 2
_NS = 16
_NW = _NC * _NS


_NRP = 10240


def _edge_chunks(src, dst, n_edges, cw, nb):
    ch = -(-n_edges // (_NW * cw))
    ch = -(-ch // nb) * nb
    ep = _NW * cw * ch
    npad = ep - n_edges
    pad_src = jnp.arange(npad, dtype=jnp.int32) % N_NODES_K
    pad_dst = N_NODES_K + (jnp.arange(npad, dtype=jnp.int32) % (_NRP - N_NODES_K))
    srcp = jnp.concatenate([src, pad_src]).reshape(_NW, ch, cw)
    dstp = jnp.concatenate([dst, pad_dst]).reshape(_NW, ch, cw)
    return srcp, dstp, ch


def _degree_sc(dstp, ch):
    mesh = plsc.VectorSubcoreMesh(core_axis_name="c", subcore_axis_name="s")
    rows_per_tile = _NRP // _NS

    @functools.partial(
        pl.kernel, mesh=mesh,
        out_type=jax.ShapeDtypeStruct((_NC, _NRP, 16), jnp.float32),
        compiler_params=pltpu.CompilerParams(use_tc_tiling_on_sc=False),
        scratch_types=[
            pltpu.VMEM((ch, 128), jnp.int32),
            pltpu.VMEM((128, 16), jnp.float32),
            pltpu.VMEM_SHARED((_NRP, 16), jnp.float32),
        ])
    def k(dstp_hbm, out_hbm, di_v, ones_v, acc_sp):
        c = lax.axis_index("c")
        s = lax.axis_index("s")
        w = c * _NS + s
        pltpu.sync_copy(dstp_hbm.at[w], di_v)

        def zrow(r, carry):
            ones_v[r, :] = jnp.zeros((16,), jnp.float32)
            return carry

        lax.fori_loop(0, 128, zrow, 0)
        for b in range(rows_per_tile // 128):
            pltpu.sync_copy(ones_v, acc_sp.at[pl.ds(s * rows_per_tile + b * 128, 128)])

        def orow(r, carry):
            ones_v[r, :] = jnp.ones((16,), jnp.float32)
            return carry

        lax.fori_loop(0, 128, orow, 0)
        plsc.subcore_barrier()

        def chunk(j, carry):
            pltpu.sync_copy(ones_v, acc_sp.at[di_v.at[j]], add=True)
            return carry

        lax.fori_loop(0, ch, chunk, 0)
        plsc.subcore_barrier()
        pltpu.sync_copy(acc_sp.at[pl.ds(s * rows_per_tile, rows_per_tile)],
                        out_hbm.at[c, pl.ds(s * rows_per_tile, rows_per_tile)])

    return k(dstp)


def _msg_pass_sc(g, srcp, dstp, ch, d, cw, nb, ovl):
    mesh = plsc.VectorSubcoreMesh(core_axis_name="c", subcore_axis_name="s")
    rows_per_tile = _NRP // _NS
    h = nb // 2

    @functools.partial(
        pl.kernel, mesh=mesh,
        out_type=jax.ShapeDtypeStruct((_NC, _NRP, d), jnp.float32),
        compiler_params=pltpu.CompilerParams(use_tc_tiling_on_sc=False),
        scratch_types=[
            pltpu.VMEM((ch, cw), jnp.int32),
            pltpu.VMEM((ch, cw), jnp.int32),
            pltpu.VMEM((nb, cw, d), jnp.float32),
            pltpu.VMEM_SHARED((_NRP, d), jnp.float32),
            pltpu.SemaphoreType.DMA,
            pltpu.SemaphoreType.DMA,
        ])
    def k(g_hbm, srcp_hbm, dstp_hbm, out_hbm, si_v, di_v, rb_v, acc_sp, sa, sb):
        c = lax.axis_index("c")
        s = lax.axis_index("s")
        w = c * _NS + s
        pltpu.sync_copy(srcp_hbm.at[w], si_v)
        pltpu.sync_copy(dstp_hbm.at[w], di_v)

        def zrow(r, carry):
            for cc in range(d // 16):
                rb_v[0, r, pl.ds(cc * 16, 16)] = jnp.zeros((16,), jnp.float32)
            return carry

        lax.fori_loop(0, cw, zrow, 0)
        base = s * rows_per_tile
        for b in range(rows_per_tile // cw):
            pltpu.sync_copy(rb_v.at[0], acc_sp.at[pl.ds(base + b * cw, cw)])
        rem = rows_per_tile % cw
        if rem:
            pltpu.sync_copy(rb_v.at[0, pl.ds(0, rem)],
                            acc_sp.at[pl.ds(base + rows_per_tile - rem, rem)])
        plsc.subcore_barrier()

        def group(t, carry):
            j = nb * t
            if ovl:
                cps_a = [pltpu.async_copy(g_hbm.at[si_v.at[j + i]], rb_v.at[i], sa)
                         for i in range(h)]
                for cp in cps_a:
                    cp.wait()
                cps_b = [pltpu.async_copy(g_hbm.at[si_v.at[j + h + i]], rb_v.at[h + i], sb)
                         for i in range(nb - h)]
                for i in range(h):
                    pltpu.sync_copy(rb_v.at[i], acc_sp.at[di_v.at[j + i]], add=True)
                for cp in cps_b:
                    cp.wait()
                for i in range(nb - h):
                    pltpu.sync_copy(rb_v.at[h + i], acc_sp.at[di_v.at[j + h + i]], add=True)
            else:
                cps = [pltpu.async_copy(g_hbm.at[si_v.at[j + i]], rb_v.at[i], sa)
                       for i in range(nb)]
                for cp in cps:
                    cp.wait()
                for i in range(nb):
                    pltpu.sync_copy(rb_v.at[i], acc_sp.at[di_v.at[j + i]], add=True)
            return carry

        lax.fori_loop(0, ch // nb, group, 0)
        plsc.subcore_barrier()
        pltpu.sync_copy(acc_sp.at[pl.ds(s * rows_per_tile, rows_per_tile)],
                        out_hbm.at[c, pl.ds(s * rows_per_tile, rows_per_tile)])

    return k(g, srcp, dstp)


def _emb_pairs_sc(acc2p, g2, dinv16, b2r, tsrc, tdst):
    q = tsrc.shape[0]
    ch = -(-q // (_NW * 128))
    qp = _NW * 128 * ch
    pad = jnp.arange(qp - q, dtype=jnp.int32) % N_NODES_K
    tsrcp = jnp.concatenate([tsrc, pad]).reshape(_NW, ch, 128)
    tdstp = jnp.concatenate([tdst, pad]).reshape(_NW, ch, 128)
    mesh = plsc.VectorSubcoreMesh(core_axis_name="c", subcore_axis_name="s")
    rpt = N_NODES_K // _NS

    @functools.partial(
        pl.kernel, mesh=mesh,
        out_type=jax.ShapeDtypeStruct((_NW, ch, 128, 16), jnp.float32),
        compiler_params=pltpu.CompilerParams(use_tc_tiling_on_sc=False),
        scratch_types=[
            pltpu.VMEM((rpt, 16), jnp.float32),
            pltpu.VMEM((rpt, 16), jnp.float32),
            pltpu.VMEM((rpt, 16), jnp.float32),
            pltpu.VMEM((rpt, 16), jnp.float32),
            pltpu.VMEM((1, 16), jnp.float32),
            pltpu.VMEM_SHARED((N_NODES_K, 16), jnp.float32),
            pltpu.VMEM((ch, 128), jnp.int32),
            pltpu.VMEM((ch, 128), jnp.int32),
            pltpu.VMEM((128, 16), jnp.float32),
            pltpu.VMEM((128, 16), jnp.float32),
            pltpu.SemaphoreType.DMA,
            pltpu.SemaphoreType.DMA,
        ])
    def k(a0_hbm, a1_hbm, g2_hbm, dv_hbm, b2_hbm, tsrc_hbm, tdst_hbm, out_hbm,
          a0_v, a1_v, g2_v, dv_v, b2_v, emb_sp, ia_v, ib_v, ra_v, rb_v, sa, sb):
        c = lax.axis_index("c")
        s = lax.axis_index("s")
        w = c * _NS + s
        base = s * rpt
        pltpu.sync_copy(a0_hbm.at[pl.ds(base, rpt)], a0_v)
        pltpu.sync_copy(a1_hbm.at[pl.ds(base, rpt)], a1_v)
        pltpu.sync_copy(g2_hbm.at[pl.ds(base, rpt)], g2_v)
        pltpu.sync_copy(dv_hbm.at[pl.ds(base, rpt)], dv_v)
        pltpu.sync_copy(b2_hbm, b2_v)
        pltpu.sync_copy(tsrc_hbm.at[w], ia_v)
        pltpu.sync_copy(tdst_hbm.at[w], ib_v)

        lanes = lax.iota(jnp.int32, 16)

        def group(gi, carry):
            r0 = gi * 16
            packed = jnp.zeros((16,), jnp.float32)
            for kk in range(16):
                r = r0 + kk
                e = dv_v[r, :] * (a0_v[r, :] + a1_v[r, :] + g2_v[r, :]) + b2_v[0, :]
                e = jnp.maximum(e, 0.0)
                a0_v[r, :] = e
                rn2 = e * e
                for st in (8, 4, 2, 1):
                    rn2 = rn2 + rn2.at[lanes ^ st].get(mode="promise_in_bounds")
                packed = jnp.where(lanes == kk, rn2, packed)
            xx = jnp.maximum(packed, 1.0)
            y = 1.0 / xx
            xh = 0.5 * xx
            for _ in range(18):
                y = y * (1.5 - xh * y * y)
            scale = jnp.where(packed > 1.0, y, 1.0)
            for kk in range(16):
                r = r0 + kk
                sk = scale.at[jnp.full((16,), kk, jnp.int32)].get(
                    mode="promise_in_bounds")
                a0_v[r, :] = a0_v[r, :] * sk
            return carry

        lax.fori_loop(0, rpt // 16, group, 0)
        e = dv_v[rpt - 1, :] * (a0_v[rpt - 1, :] + a1_v[rpt - 1, :]
                                + g2_v[rpt - 1, :]) + b2_v[0, :]
        e = jnp.maximum(e, 0.0)
        rn2 = e * e
        for st in (8, 4, 2, 1):
            rn2 = rn2 + rn2.at[lanes ^ st].get(mode="promise_in_bounds")
        xx = jnp.maximum(rn2, 1.0)
        y = 1.0 / xx
        xh = 0.5 * xx
        for _ in range(18):
            y = y * (1.5 - xh * y * y)
        a0_v[rpt - 1, :] = e * jnp.where(rn2 > 1.0, y, 1.0)
        pltpu.sync_copy(a0_v, emb_sp.at[pl.ds(base, rpt)])
        plsc.subcore_barrier()

        def chunk(j, carry):
            ca = pltpu.async_copy(emb_sp.at[ia_v.at[j]], ra_v, sa)
            cb = pltpu.async_copy(emb_sp.at[ib_v.at[j]], rb_v, sb)
            ca.wait()
            cb.wait()

            def prow(r, carry2):
                d = ra_v[r, :] - rb_v[r, :]
                ra_v[r, :] = d * d
                return carry2

            lax.fori_loop(0, 128, prow, 0)
            pltpu.sync_copy(ra_v, out_hbm.at[w, j])
            return carry

        lax.fori_loop(0, ch, chunk, 0)

    out = k(acc2p[0], acc2p[1], g2, dinv16, b2r, tsrcp, tdstp)
    return out.reshape(qp, 16)[:q]


def _decode_mlp_body(sq_ref, pi_ref, l1Wa_ref, l1Wb_ref, l1b_ref, lW_ref, lb_ref, out_ref):
    sq = sq_ref[...]
    pi = pi_ref[...]
    z = (jnp.dot(sq, l1Wa_ref[...], preferred_element_type=jnp.float32)
         + jnp.dot(pi, l1Wb_ref[...], preferred_element_type=jnp.float32)
         + l1b_ref[...])
    z = jnp.where(z >= 0, z, 0.2 * z)
    s = jnp.abs(jnp.dot(z, lW_ref[...], preferred_element_type=jnp.float32) + lb_ref[...])
    s = jnp.clip(s, 0.0, 40.0)
    out_ref[...] = 1.0 / (jnp.exp(s - 2.0) + 1.0)


def _decode_mlp(sqdist, PI_edges, l1W, l1b, lW, lb):
    q = sqdist.shape[0]
    blk = 5000
    grid = (q // blk,)
    out = pl.pallas_call(
        _decode_mlp_body,
        grid=grid,
        in_specs=[
            pl.BlockSpec((blk, 16), lambda i: (i, 0)),
            pl.BlockSpec((blk, 25), lambda i: (i, 0)),
            pl.BlockSpec((16, 25), lambda i: (0, 0)),
            pl.BlockSpec((25, 25), lambda i: (0, 0)),
            pl.BlockSpec((1, 25), lambda i: (0, 0)),
            pl.BlockSpec((25, 1), lambda i: (0, 0)),
            pl.BlockSpec((1, 1), lambda i: (0, 0)),
        ],
        out_specs=pl.BlockSpec((blk, 1), lambda i: (i, 0)),
        out_shape=jax.ShapeDtypeStruct((q, 1), jnp.float32),
    )(sqdist, PI_edges, l1W[:16], l1W[16:], l1b.reshape(1, 25), lW, lb.reshape(1, 1))
    return out.reshape(-1)


_RB = 1000


def _mm1_body(x_ref, W1p_ref, h_ref):
    h_ref[...] = jnp.dot(x_ref[...], W1p_ref[...],
                         preferred_element_type=jnp.float32)


def _mm1_tc(x, W1p):
    grid = (N_NODES_K // _RB,)
    return pl.pallas_call(
        _mm1_body,
        grid=grid,
        in_specs=[
            pl.BlockSpec((_RB, 128), lambda i: (i, 0)),
            pl.BlockSpec((128, 112), lambda i: (0, 0)),
        ],
        out_specs=pl.BlockSpec((_RB, 112), lambda i: (i, 0)),
        out_shape=jax.ShapeDtypeStruct((N_NODES_K, 112), jnp.float32),
    )(x, W1p)


def _prep_body(h_ref, dg0_ref, dg1_ref, g1_ref, dinv_ref, dinv16_ref):
    deg = dg0_ref[...][:, 0:1] + dg1_ref[...][:, 0:1] + 1.0
    dv = jax.lax.rsqrt(jnp.maximum(deg, 1e-12))
    g1_ref[...] = h_ref[...] * dv
    dinv_ref[...] = dv
    dinv16_ref[...] = jnp.broadcast_to(dv, (dv.shape[0], 16))


def _prep_tc(h1pre, degp):
    grid = (N_NODES_K // _RB,)
    return pl.pallas_call(
        _prep_body,
        grid=grid,
        in_specs=[
            pl.BlockSpec((_RB, 112), lambda i: (i, 0)),
            pl.BlockSpec((_RB, 16), lambda i: (i, 0)),
            pl.BlockSpec((_RB, 16), lambda i: (i, 0)),
        ],
        out_specs=[
            pl.BlockSpec((_RB, 112), lambda i: (i, 0)),
            pl.BlockSpec((_RB, 1), lambda i: (i, 0)),
            pl.BlockSpec((_RB, 16), lambda i: (i, 0)),
        ],
        out_shape=[
            jax.ShapeDtypeStruct((N_NODES_K, 112), jnp.float32),
            jax.ShapeDtypeStruct((N_NODES_K, 1), jnp.float32),
            jax.ShapeDtypeStruct((N_NODES_K, 16), jnp.float32),
        ],
    )(h1pre, degp[0], degp[1])


def _comb1_body(a0_ref, a1_ref, g1_ref, dinv_ref, b1p_ref, W2p_ref, g2_ref):
    dv = dinv_ref[...]
    h1 = dv * (a0_ref[...] + a1_ref[...] + g1_ref[...]) + b1p_ref[...]
    h1 = jnp.maximum(h1, 0.0)
    g2_ref[...] = jnp.dot(h1, W2p_ref[...],
                          preferred_element_type=jnp.float32) * dv


def _comb1_tc(accp, g1, dinv, b1p, W2p):
    grid = (N_NODES_K // _RB,)
    return pl.pallas_call(
        _comb1_body,
        grid=grid,
        in_specs=[
            pl.BlockSpec((_RB, 112), lambda i: (i, 0)),
            pl.BlockSpec((_RB, 112), lambda i: (i, 0)),
            pl.BlockSpec((_RB, 112), lambda i: (i, 0)),
            pl.BlockSpec((_RB, 1), lambda i: (i, 0)),
            pl.BlockSpec((1, 112), lambda i: (0, 0)),
            pl.BlockSpec((112, 16), lambda i: (0, 0)),
        ],
        out_specs=pl.BlockSpec((_RB, 16), lambda i: (i, 0)),
        out_shape=jax.ShapeDtypeStruct((N_NODES_K, 16), jnp.float32),
    )(accp[0], accp[1], g1, dinv, b1p, W2p)


def kernel(x, edge_index, total_edges, PI_edges, edges_y, W1, b1, W2, b2, l1W, l1b, lW, lb):
    src = jnp.asarray(edge_index[0], jnp.int32)
    dst = jnp.asarray(edge_index[1], jnp.int32)
    srcp1, dstp1, ch1 = _edge_chunks(src, dst, N_EDGES_K, 56, 6)
    srcp2, dstp2, ch2 = _edge_chunks(src, dst, N_EDGES_K, 128, 8)
    degp = _degree_sc(dstp2, ch2)
    W1p = jnp.pad(W1, ((0, 0), (0, 12)))
    b1p = jnp.pad(b1, (0, 12)).reshape(1, 112)
    W2p = jnp.pad(W2, ((0, 12), (0, 0)))
    h1pre = _mm1_tc(x, W1p)
    g1, dinv, dinv16 = _prep_tc(h1pre, degp)
    acc1p = _msg_pass_sc(g1, srcp1, dstp1, ch1, 112, 56, 6, True)
    g2 = _comb1_tc(acc1p, g1, dinv, b1p, W2p)
    acc2p = _msg_pass_sc(g2, srcp2, dstp2, ch2, 16, 128, 8, False)
    tsrc = jnp.asarray(total_edges[:, 0], jnp.int32)
    tdst = jnp.asarray(total_edges[:, 1], jnp.int32)
    sqdist = _emb_pairs_sc(acc2p, g2, dinv16, b2.reshape(1, 16), tsrc, tdst)
    prob = _decode_mlp(sqdist, PI_edges, l1W, l1b, lW, lb)
    return (prob, edges_y)
